# baseline (speedup 1.0000x reference)
import functools

import jax
import jax.numpy as jnp
from jax import lax
from jax.experimental import pallas as pl
from jax.experimental.pallas import tpu as pltpu
from jax.experimental.pallas import tpu_sc as plsc

B = 64
T = 512
TEXT_VOCAB = 256
D_MODEL = 128
N_EMOTIONS = 8
MEL_DIM = 80
NE = N_EMOTIONS * TEXT_VOCAB

N_MC = 2
MC = MEL_DIM // N_MC
N_BK = 16
BK = B // N_BK


def _tables_body(text_ref, emo_ref, emo_tab_ref, text_tab_ref, w_ref, b_ref,
                 bigt_ref, idx_ref):
    tpt = lax.dot_general(w_ref[...], text_tab_ref[...],
                          (((0,), (1,)), ((), ())),
                          preferred_element_type=jnp.float32)
    ept = lax.dot_general(w_ref[...], emo_tab_ref[...],
                          (((0,), (1,)), ((), ())),
                          preferred_element_type=jnp.float32)
    tpt = tpt + b_ref[...]
    bigt_ref[...] = ept[:, :, None] + tpt[:, None, :]
    text = text_ref[...]
    idx_ref[...] = ((text >> 7) * (N_EMOTIONS * 128)
                    + emo_ref[...] * 128 + (text & 127))


def _build_tables(text_tensor, emotion_label, emotion_table, text_table,
                  W_out, b_out):
    return pl.pallas_call(
        _tables_body,
        out_shape=(
            jax.ShapeDtypeStruct((MEL_DIM, N_EMOTIONS, TEXT_VOCAB), jnp.float32),
            jax.ShapeDtypeStruct((B, T), jnp.int32),
        ),
    )(text_tensor, emotion_label.reshape(B, 1), emotion_table, text_table,
      W_out, b_out.reshape(MEL_DIM, 1))


def _sc_gather(idx, table):
    info = plsc.get_sparse_core_info()
    nc, ns = info.num_cores, info.num_subcores
    assert nc * ns == N_MC * N_BK
    mesh = plsc.VectorSubcoreMesh(core_axis_name="c", subcore_axis_name="s")

    n_ch = MC // 8
    ch_w = 8 * NE

    @functools.partial(
        pl.kernel,
        out_type=jax.ShapeDtypeStruct((B * MEL_DIM * T,), jnp.float32),
        mesh=mesh,
        scratch_types=[
            pltpu.VMEM((8 * T,), jnp.int32),
            pltpu.VMEM((ch_w,), jnp.float32),
            pltpu.VMEM((ch_w,), jnp.float32),
            pltpu.VMEM((8 * T,), jnp.float32),
            pltpu.VMEM((8 * T,), jnp.float32),
            pltpu.VMEM((8 * T,), jnp.float32),
            pltpu.VMEM((8 * T,), jnp.float32),
            pltpu.SemaphoreType.DMA,
            pltpu.SemaphoreType.DMA,
            pltpu.SemaphoreType.DMA,
        ],
        compiler_params=pltpu.CompilerParams(needs_layout_passes=False),
    )
    def gather_kernel(idx_hbm, table_hbm, out_hbm, idx_v, tab_a, tab_b,
                      sl0, sl1, sl2, sl3, isem, tsem, osem):
        wid = lax.axis_index("s") * nc + lax.axis_index("c")
        c2 = wid % N_MC
        k = wid // N_MC
        tr = k // 2
        rq = k % 2
        tbase = c2 * MC * NE
        tabs = [tab_a, tab_b]
        tcp = [pltpu.async_copy(table_hbm.at[pl.ds(tbase + r * ch_w, ch_w)],
                                tabs[r % 2], tsem) for r in range(2)]
        icopy = pltpu.async_copy(idx_hbm.at[pl.ds(tr * 4096, 4096)], idx_v,
                                 isem)
        icopy.wait()

        def fill(slab, tab, bl):
            @plsc.parallel_loop(0, 8, unroll=2)
            def per_gl(gl):
                for tc in range(4):
                    iv = idx_v[pl.ds(tc * 1024 + (rq * BK + bl) * 128
                                     + gl * 16, 16)]
                    vals = [plsc.load_gather(tab, [iv + j * NE])
                            for j in range(8)]
                    for j, v in enumerate(vals):
                        slab[pl.ds(tc * 1024 + j * 128 + gl * 16, 16)] = v

        slabs = [sl0, sl1, sl2, sl3]
        stores = [None] * (n_ch * BK)
        for r in range(n_ch):
            tcp[r % 2].wait()
            tab = tabs[r % 2]
            for bl in range(BK):
                s = r * BK + bl
                if s >= 4:
                    stores[s - 4].wait()
                slab = slabs[s % 4]
                fill(slab, tab, bl)
                b = 8 * tr + rq * BK + bl
                dst = out_hbm.at[pl.ds(b * (MEL_DIM * T) + c2 * (MC * T)
                                       + r * (8 * T), 8 * T)]
                stores[s] = pltpu.async_copy(slab, dst, osem)
            if r + 2 < n_ch:
                tcp[r % 2] = pltpu.async_copy(
                    table_hbm.at[pl.ds(tbase + (r + 2) * ch_w, ch_w)],
                    tabs[r % 2], tsem)
        for s in range(n_ch * BK - 4, n_ch * BK):
            stores[s].wait()

    return gather_kernel(idx, table)


def kernel(text_tensor, emotion_label, emotion_table, text_table, W_out, b_out):
    bigt, idx = _build_tables(text_tensor, emotion_label, emotion_table,
                              text_table, W_out, b_out)
    tab1d = (bigt.reshape(MEL_DIM, N_EMOTIONS, 2, 128)
             .transpose(0, 2, 1, 3).reshape(MEL_DIM * NE))
    idx1d = idx.reshape(8, 8, 4, 128).transpose(0, 2, 1, 3).reshape(B * T)
    out = _sc_gather(idx1d, tab1d)
    return (out.reshape(B, MEL_DIM // 8, T // 128, 8, 128)
            .transpose(0, 2, 4, 1, 3).reshape(B, T, MEL_DIM))


# speedup vs baseline: 1.1542x; 1.1542x over previous
import functools

import jax
import jax.numpy as jnp
from jax import lax
from jax.experimental import pallas as pl
from jax.experimental.pallas import tpu as pltpu
from jax.experimental.pallas import tpu_sc as plsc

B = 64
T = 512
TEXT_VOCAB = 256
D_MODEL = 128
N_EMOTIONS = 8
MEL_DIM = 80
NE = N_EMOTIONS * TEXT_VOCAB

N_MC = 2
MC = MEL_DIM // N_MC
N_BK = 16
BK = B // N_BK


def _tables_body(text_ref, emo_ref, emo_tab_ref, text_tab_ref, w_ref, b_ref,
                 bigt_ref, idx_ref):
    tpt = lax.dot_general(w_ref[...], text_tab_ref[...],
                          (((0,), (1,)), ((), ())),
                          preferred_element_type=jnp.float32)
    ept = lax.dot_general(w_ref[...], emo_tab_ref[...],
                          (((0,), (1,)), ((), ())),
                          preferred_element_type=jnp.float32)
    tpt = tpt + b_ref[...]
    bigt_ref[...] = ept[:, :, None] + tpt[:, None, :]
    text = text_ref[...]
    idx_ref[...] = ((text >> 7) * (N_EMOTIONS * 128)
                    + emo_ref[...] * 128 + (text & 127))


def _build_tables(text_tensor, emotion_label, emotion_table, text_table,
                  W_out, b_out):
    return pl.pallas_call(
        _tables_body,
        out_shape=(
            jax.ShapeDtypeStruct((MEL_DIM, N_EMOTIONS, TEXT_VOCAB), jnp.float32),
            jax.ShapeDtypeStruct((B, T), jnp.int32),
        ),
    )(text_tensor, emotion_label.reshape(B, 1), emotion_table, text_table,
      W_out, b_out.reshape(MEL_DIM, 1))


def _sc_gather(idx, table):
    info = plsc.get_sparse_core_info()
    nc, ns = info.num_cores, info.num_subcores
    assert nc * ns == N_MC * N_BK
    mesh = plsc.VectorSubcoreMesh(core_axis_name="c", subcore_axis_name="s")

    @functools.partial(
        pl.kernel,
        out_type=jax.ShapeDtypeStruct((B * MEL_DIM * T,), jnp.float32),
        mesh=mesh,
        scratch_types=[
            pltpu.VMEM((8 * T,), jnp.int32),
            pltpu.VMEM((MC * NE,), jnp.float32),
            pltpu.VMEM((MC * T,), jnp.float32),
            pltpu.VMEM((MC * T,), jnp.float32),
            pltpu.SemaphoreType.DMA,
            pltpu.SemaphoreType.DMA,
        ],
        compiler_params=pltpu.CompilerParams(needs_layout_passes=False),
    )
    def gather_kernel(idx_hbm, table_hbm, out_hbm, idx_v, tab_v, slab_a,
                      slab_b, isem, osem):
        wid = lax.axis_index("s") * nc + lax.axis_index("c")
        c2 = wid % N_MC
        k = wid // N_MC
        tr = k // 2
        rq = k % 2
        tcopy = pltpu.async_copy(table_hbm.at[pl.ds(c2 * MC * NE, MC * NE)],
                                 tab_v, osem)
        icopy = pltpu.async_copy(idx_hbm.at[pl.ds(tr * 4096, 4096)], idx_v,
                                 isem)
        icopy.wait()
        tcopy.wait()

        def fill(slab, bl):
            @plsc.parallel_loop(0, 8, unroll=2)
            def per_gl(gl):
                for tc in range(4):
                    iv = idx_v[pl.ds(tc * 1024 + (rq * BK + bl) * 128
                                     + gl * 16, 16)]
                    for m0 in range(0, MC, 8):
                        vals = [plsc.load_gather(tab_v, [iv + (m0 + j) * NE])
                                for j in range(8)]
                        for j, v in enumerate(vals):
                            m = m0 + j
                            slab[pl.ds((m // 8) * 4096 + tc * 1024
                                       + (m % 8) * 128 + gl * 16, 16)] = v

        stores = [None] * BK
        for bl in range(BK):
            if bl >= 2:
                stores[bl - 2].wait()
            slab = slab_a if bl % 2 == 0 else slab_b
            fill(slab, bl)
            b = 8 * tr + rq * BK + bl
            dst = out_hbm.at[pl.ds(b * (MEL_DIM * T) + c2 * (MC * T), MC * T)]
            stores[bl] = pltpu.async_copy(slab, dst, osem)
        for bl in range(BK - 2, BK):
            stores[bl].wait()

    return gather_kernel(idx, table)


def kernel(text_tensor, emotion_label, emotion_table, text_table, W_out, b_out):
    bigt, idx = _build_tables(text_tensor, emotion_label, emotion_table,
                              text_table, W_out, b_out)
    tab1d = (bigt.reshape(MEL_DIM, N_EMOTIONS, 2, 128)
             .transpose(0, 2, 1, 3).reshape(MEL_DIM * NE))
    idx1d = idx.reshape(8, 8, 4, 128).transpose(0, 2, 1, 3).reshape(B * T)
    out = _sc_gather(idx1d, tab1d)
    return (out.reshape(B, MEL_DIM // 8, T // 128, 8, 128)
            .transpose(0, 2, 4, 1, 3).reshape(B, T, MEL_DIM))
